# Initial kernel scaffold; baseline (speedup 1.0000x reference)
import jax
import jax.numpy as jnp
from jax.experimental import pallas as pl


def kernel(p_node_feat, p_edge_index, r_node_feat, r_edge_index, batch, Wr, br, Wlin, blin):
    raise NotImplementedError("write your pallas kernel here")


# speedup vs baseline: 34.1069x; 34.1069x over previous
import jax
import jax.numpy as jnp
from jax import lax
from jax.experimental import pallas as pl
from jax.experimental.pallas import tpu as pltpu
from jax.experimental.pallas import tpu_sc as plsc

N = 10000
E = 640000
F = 120
H = 128
HH = H // 2
NC = 2
NS = 16
NW = NC * NS
K = 80

EPW = E // NW
NCHD = EPW // K

EPS = E // NS
NCHA = EPS // K

BLK = 400

_mesh = plsc.VectorSubcoreMesh(
    core_axis_name="c", subcore_axis_name="s", num_cores=NC, num_subcores=NS)


def _deg_body(dst3, zeros_n, deg_out, dst_buf, ones_buf, deg_sh):
    c = lax.axis_index("c")
    s = lax.axis_index("s")
    wid = c * NS + s
    pltpu.sync_copy(dst3.at[wid], dst_buf)
    for j in range(K // 16):
        ones_buf[pl.ds(j * 16, 16)] = jnp.ones((16,), jnp.float32)

    @pl.when(s == 0)
    def _():
        pltpu.sync_copy(zeros_n, deg_sh)

    plsc.subcore_barrier()

    @pl.loop(0, NCHD)
    def _chunk(cid):
        pltpu.sync_copy(ones_buf, deg_sh.at[dst_buf.at[cid]], add=True)

    plsc.subcore_barrier()

    @pl.when(s == 0)
    def _():
        pltpu.sync_copy(deg_sh, deg_out.at[c])


def _agg_body(src3, dst3, g0, g1, agg_out,
              src_buf, dst_buf, rows0, rows1, sem0, sem1, agg_sh):
    c = lax.axis_index("c")
    s = lax.axis_index("s")
    pltpu.sync_copy(src3.at[s], src_buf)
    pltpu.sync_copy(dst3.at[s], dst_buf)

    @pl.when(s == 0)
    def _():
        @pl.when(c == 0)
        def _():
            pltpu.sync_copy(g0, agg_sh)

        @pl.when(c > 0)
        def _():
            pltpu.sync_copy(g1, agg_sh)

    plsc.subcore_barrier()

    def run(g_in):
        pltpu.async_copy(g_in.at[src_buf.at[0]], rows0, sem0)
        pltpu.async_copy(g_in.at[src_buf.at[1]], rows1, sem1)

        @pl.loop(0, NCHA, step=2)
        def _chunk(base):
            for b, rows, sem in ((0, rows0, sem0), (1, rows1, sem1)):
                cid = base + b
                pltpu.make_async_copy(g_in.at[src_buf.at[cid]], rows, sem).wait()
                pltpu.sync_copy(rows, agg_sh.at[dst_buf.at[cid]], add=True)

                @pl.when(cid + 2 < NCHA)
                def _():
                    pltpu.async_copy(g_in.at[src_buf.at[cid + 2]], rows, sem)

    @pl.when(c == 0)
    def _():
        run(g0)

    @pl.when(c > 0)
    def _():
        run(g1)

    plsc.subcore_barrier()
    off = pl.multiple_of(s * 624, 8)

    @pl.when(s < NS - 1)
    def _():
        pltpu.sync_copy(agg_sh.at[pl.ds(off, 624)],
                        agg_out.at[c, pl.ds(off, 624)])

    @pl.when(s == NS - 1)
    def _():
        pltpu.sync_copy(agg_sh.at[pl.ds(15 * 624, 640)],
                        agg_out.at[c, pl.ds(15 * 624, 640)])


def _scale_body(x_ref, w_ref, degt_ref, g_ref):
    deg = degt_ref[:, 0:1] + degt_ref[:, 1:2] + 1.0
    dinv = lax.rsqrt(deg)
    h = jnp.dot(x_ref[...], w_ref[...],
                preferred_element_type=jnp.float32,
                precision=lax.Precision.HIGHEST)
    g_ref[...] = h * dinv


def _out_body(agg_ref, degt_ref, br_ref, wl_ref, bl_ref, o_ref):
    a = jnp.concatenate([agg_ref[0], agg_ref[1]], axis=1)
    deg = degt_ref[:, 0:1] + degt_ref[:, 1:2] + 1.0
    dinv = lax.rsqrt(deg)
    v = jnp.maximum(a * dinv + br_ref[...][None, :], 0.0)
    o_ref[...] = jnp.dot(v, wl_ref[...],
                         preferred_element_type=jnp.float32,
                         precision=lax.Precision.HIGHEST) + bl_ref[...][None, :]


def kernel(p_node_feat, p_edge_index, r_node_feat, r_edge_index, batch,
           Wr, br, Wlin, blin):
    src = r_edge_index[0].astype(jnp.int32)
    dst = r_edge_index[1].astype(jnp.int32)
    src16 = src.reshape(NS, NCHA, K)
    dst16 = dst.reshape(NS, NCHA, K)
    dst32 = dst.reshape(NW, NCHD, K)
    zeros_n = jnp.zeros((N,), jnp.float32)

    deg = pl.kernel(
        _deg_body,
        out_type=jax.ShapeDtypeStruct((NC, N), jnp.float32),
        mesh=_mesh,
        scratch_types=[
            pltpu.VMEM((NCHD, K), jnp.int32),
            pltpu.VMEM((K,), jnp.float32),
            pltpu.VMEM_SHARED((N,), jnp.float32),
        ],
    )(dst32, zeros_n)
    degt = deg.T

    g = pl.pallas_call(
        _scale_body,
        grid=(N // BLK,),
        in_specs=[
            pl.BlockSpec((BLK, F), lambda i: (i, 0)),
            pl.BlockSpec((F, H), lambda i: (0, 0)),
            pl.BlockSpec((BLK, NC), lambda i: (i, 0)),
        ],
        out_specs=pl.BlockSpec((BLK, H), lambda i: (i, 0)),
        out_shape=jax.ShapeDtypeStruct((N, H), jnp.float32),
    )(r_node_feat, Wr, degt)
    g0 = g[:, :HH]
    g1 = g[:, HH:]

    agg = pl.kernel(
        _agg_body,
        out_type=jax.ShapeDtypeStruct((NC, N, HH), jnp.float32),
        mesh=_mesh,
        compiler_params=pltpu.CompilerParams(use_tc_tiling_on_sc=False),
        scratch_types=[
            pltpu.VMEM((NCHA, K), jnp.int32),
            pltpu.VMEM((NCHA, K), jnp.int32),
            pltpu.VMEM((K, HH), jnp.float32),
            pltpu.VMEM((K, HH), jnp.float32),
            pltpu.SemaphoreType.DMA,
            pltpu.SemaphoreType.DMA,
            pltpu.VMEM_SHARED((N, HH), jnp.float32),
        ],
    )(src16, dst16, g0, g1)

    wl = jnp.zeros((H, H), jnp.float32).at[:, :2].set(Wlin)
    bl = jnp.zeros((H,), jnp.float32).at[:2].set(blin)

    out = pl.pallas_call(
        _out_body,
        grid=(N // BLK,),
        in_specs=[
            pl.BlockSpec((NC, BLK, HH), lambda i: (0, i, 0)),
            pl.BlockSpec((BLK, NC), lambda i: (i, 0)),
            pl.BlockSpec((H,), lambda i: (0,)),
            pl.BlockSpec((H, H), lambda i: (0, 0)),
            pl.BlockSpec((H,), lambda i: (0,)),
        ],
        out_specs=pl.BlockSpec((BLK, H), lambda i: (i, 0)),
        out_shape=jax.ShapeDtypeStruct((N, H), jnp.float32),
    )(agg, degt, br, wl, bl)

    return out[:, :2]
